# baseline (speedup 1.0000x reference)
import functools

import jax
import jax.numpy as jnp
from jax import lax
from jax.experimental import pallas as pl
from jax.experimental.pallas import tpu as pltpu
from jax.experimental.pallas import tpu_sc as plsc

N = 10000
E = 320000
C = 128
IC = 256
CH = C // 2

NC = 2
NS = 16
NW = NC * NS
WIN = 80

EPW_D = E // NW
NWIN_D = EPW_D // WIN

EPW_P = E // NS
WINP = 128
NFULL = EPW_P // WINP
TAIL = EPW_P - NFULL * WINP
NBUF = 3
NGRP = NFULL // NBUF

_MESH = plsc.VectorSubcoreMesh(
    core_axis_name="c", subcore_axis_name="s", num_cores=NC, num_subcores=NS)

_Z16 = functools.partial(jnp.zeros, (16,), jnp.float32)
_Z32B = functools.partial(jnp.zeros, (32,), jnp.bfloat16)


def _deg_body(dst_hbm, out_hbm, idxb, ones_v, zb, deg_sh):
    cid = lax.axis_index("c")
    sid = lax.axis_index("s")
    wid = cid * NS + sid

    def fill(i, _):
        ones_v[pl.ds(i * 16, 16)] = _Z16() + 1.0
        return 0
    lax.fori_loop(0, WIN // 16, fill, 0)

    def fillz(i, _):
        zb[pl.ds(i * 16, 16)] = _Z16()
        return 0
    lax.fori_loop(0, 2000 // 16, fillz, 0)

    @pl.when(sid < 5)
    def _():
        pltpu.sync_copy(zb, deg_sh.at[pl.ds(sid * 2000, 2000)])

    pltpu.sync_copy(dst_hbm.at[wid], idxb)
    plsc.subcore_barrier()

    def body(w, _):
        pltpu.sync_copy(ones_v, deg_sh.at[idxb.at[w]], add=True)
        return 0
    lax.fori_loop(0, NWIN_D, body, 0)

    plsc.subcore_barrier()

    @pl.when(sid < 5)
    def _():
        pltpu.sync_copy(deg_sh.at[pl.ds(sid * 2000, 2000)], zb)
        pltpu.sync_copy(zb, out_hbm.at[pl.ds(cid * N + sid * 2000, 2000)])


_deg_kernel = pl.kernel(
    _deg_body,
    out_type=jax.ShapeDtypeStruct((NC * N,), jnp.float32),
    mesh=_MESH,
    scratch_types=[
        pltpu.VMEM((NWIN_D, WIN), jnp.int32),
        pltpu.VMEM((WIN,), jnp.float32),
        pltpu.VMEM((2000,), jnp.float32),
        pltpu.VMEM_SHARED((N,), jnp.float32),
    ],
)


def _prop_body(u_hbm, es_hbm, out_hbm, srcb, dstb, rowsbf, acc0, acc1, acc2,
               *sems):
    accs = (acc0, acc1, acc2)
    gsem = sems[:NBUF]
    ssem = sems[NBUF:]
    cid = lax.axis_index("c")
    sid = lax.axis_index("s")

    def zfill(i, _):
        for cc in range(CH // 32):
            rowsbf[0, i, pl.ds(cc * 32, 32)] = _Z32B()
        return 0
    lax.fori_loop(0, WINP, zfill, 0)

    for a in range(NBUF):
        for k in range(5):
            c = sid + k * NS

            @pl.when(c < 78)
            def _():
                pltpu.sync_copy(rowsbf.at[0],
                                accs[a].at[pl.ds(c * WINP, WINP)])

        @pl.when(sid == 0)
        def _():
            pltpu.sync_copy(rowsbf.at[0, pl.ds(0, 16)],
                            accs[a].at[pl.ds(9984, 16)])

    e0 = sid * EPW_P
    pltpu.sync_copy(es_hbm.at[0, pl.ds(e0, EPW_P)], srcb)
    pltpu.sync_copy(es_hbm.at[1, pl.ds(e0, EPW_P)], dstb)
    plsc.subcore_barrier()

    uh = u_hbm.at[cid]

    def gather(w, b, n=WINP):
        pltpu.async_copy(uh.at[srcb.at[pl.ds(w * WINP, n)]],
                         rowsbf.at[b, pl.ds(0, n)] if n != WINP
                         else rowsbf.at[b],
                         gsem[b])

    def gwait(w, b, n=WINP):
        pltpu.make_async_copy(uh.at[srcb.at[pl.ds(w * WINP, n)]],
                              rowsbf.at[b, pl.ds(0, n)] if n != WINP
                              else rowsbf.at[b], gsem[b]).wait()

    def scatter(w, b, n=WINP):
        pltpu.async_copy(rowsbf.at[b, pl.ds(0, n)] if n != WINP
                         else rowsbf.at[b],
                         accs[b].at[dstb.at[pl.ds(w * WINP, n)]], ssem[b],
                         add=True)

    def swait(w, b, n=WINP):
        pltpu.make_async_copy(rowsbf.at[b, pl.ds(0, n)] if n != WINP
                              else rowsbf.at[b],
                              accs[b].at[dstb.at[pl.ds(w * WINP, n)]],
                              ssem[b]).wait()

    for b in range(NBUF):
        gather(b, b)

    def group(g, _):
        for b in range(NBUF):
            w = g * NBUF + b
            gwait(w, b)
            scatter(w, b)

            @pl.when(w + NBUF < NFULL)
            def _():
                swait(w, b)
                gather(w + NBUF, b)
        return 0

    lax.fori_loop(0, NGRP, group, 0)

    swait(NFULL - NBUF, 0)
    gather(NFULL, 0, TAIL)
    gwait(NFULL, 0, TAIL)
    scatter(NFULL, 0, TAIL)
    swait(NFULL, 0, TAIL)
    for b in range(1, NBUF):
        swait(NFULL - NBUF + b, b)

    plsc.subcore_barrier()
    for a in range(NBUF):
        for k in range(5):
            c = sid + k * NS

            @pl.when(c < 78)
            def _():
                pltpu.sync_copy(accs[a].at[pl.ds(c * WINP, WINP)],
                                rowsbf.at[a])
                pltpu.sync_copy(rowsbf.at[a],
                                out_hbm.at[cid, a, pl.ds(c * WINP, WINP)])

        @pl.when(sid == 0)
        def _():
            pltpu.sync_copy(accs[a].at[pl.ds(9984, 16)],
                            rowsbf.at[a, pl.ds(0, 16)])
            pltpu.sync_copy(rowsbf.at[a, pl.ds(0, 16)],
                            out_hbm.at[cid, a, pl.ds(9984, 16)])


_prop_kernel = pl.kernel(
    _prop_body,
    out_type=jax.ShapeDtypeStruct((NC, NBUF, N, CH), jnp.bfloat16),
    mesh=_MESH,
    scratch_types=[
        pltpu.VMEM((EPW_P,), jnp.int32),
        pltpu.VMEM((EPW_P,), jnp.int32),
        pltpu.VMEM((NBUF, WINP, CH), jnp.bfloat16),
        pltpu.VMEM_SHARED((N, CH), jnp.bfloat16),
        pltpu.VMEM_SHARED((N, CH), jnp.bfloat16),
        pltpu.VMEM_SHARED((N, CH), jnp.bfloat16),
    ] + [pltpu.SemaphoreType.DMA] * (2 * NBUF),
    compiler_params=pltpu.CompilerParams(use_tc_tiling_on_sc=False),
)


def _pre_body(degp_ref, x_ref, dinv_ref, invdeg_ref, u1_ref):
    deg = degp_ref[0] + degp_ref[1] + 1.0
    dinv = lax.rsqrt(deg)
    dinv_ref[...] = dinv
    invdeg_ref[...] = 1.0 / deg
    u1 = (x_ref[...] * dinv).astype(jnp.bfloat16)
    u1_ref[0] = u1[:, :CH]
    u1_ref[1] = u1[:, CH:]


def _mid_body(agg_ref, x_ref, dinv_ref, invdeg_ref, w1_ref, g1_ref, b1_ref,
              w2_ref, t_ref, u2_ref):
    ap = agg_ref[...].astype(jnp.float32).sum(axis=1)
    agg = jnp.concatenate([ap[0], ap[1]], axis=1)
    s1 = agg * dinv_ref[...] + x_ref[...] * invdeg_ref[...]
    h1 = jnp.dot(s1, w1_ref[...], preferred_element_type=jnp.float32)
    m = jnp.mean(h1, axis=0, keepdims=True)
    d = h1 - m
    v = jnp.mean(d * d, axis=0, keepdims=True)
    hn = jnp.maximum(g1_ref[...] * d / jnp.sqrt(v + 1e-5) + b1_ref[...], 0.0)
    t = jnp.dot(hn, w2_ref[...], preferred_element_type=jnp.float32)
    t_ref[...] = t
    u2 = (t * dinv_ref[...]).astype(jnp.bfloat16)
    u2_ref[0] = u2[:, :CH]
    u2_ref[1] = u2[:, CH:]


def _post_body(agg_ref, t_ref, x_ref, dinv_ref, invdeg_ref, g2_ref, b2_ref,
               out_ref):
    ap = agg_ref[...].astype(jnp.float32).sum(axis=1)
    agg = jnp.concatenate([ap[0], ap[1]], axis=1)
    s2 = agg * dinv_ref[...] + t_ref[...] * invdeg_ref[...]
    m = jnp.mean(s2, axis=0, keepdims=True)
    d = s2 - m
    v = jnp.mean(d * d, axis=0, keepdims=True)
    bn = g2_ref[...] * d / jnp.sqrt(v + 1e-5) + b2_ref[...]
    out_ref[...] = jnp.maximum(bn + x_ref[...], 0.0)


def _tc_call(body, out_shapes):
    return pl.pallas_call(body, out_shape=out_shapes)


def kernel(x, es, W1, W2, g1, b1, g2, b2):
    esd = es.reshape(2, NW, NWIN_D, WIN)

    degp = _deg_kernel(esd[1])
    degp = degp.reshape(NC, N, 1)

    f32 = jnp.float32
    bf16 = jnp.bfloat16
    dinv, invdeg, u1 = _tc_call(_pre_body, [
        jax.ShapeDtypeStruct((N, 1), f32),
        jax.ShapeDtypeStruct((N, 1), f32),
        jax.ShapeDtypeStruct((NC, N, CH), bf16),
    ])(degp, x)

    agg1 = _prop_kernel(u1, es)

    t, u2 = _tc_call(_mid_body, [
        jax.ShapeDtypeStruct((N, C), f32),
        jax.ShapeDtypeStruct((NC, N, CH), bf16),
    ])(agg1, x, dinv, invdeg, W1, g1.reshape(1, IC), b1.reshape(1, IC), W2)

    agg2 = _prop_kernel(u2, es)

    out = _tc_call(_post_body, jax.ShapeDtypeStruct((N, C), f32))(
        agg2, t, x, dinv, invdeg, g2.reshape(1, C), b2.reshape(1, C))
    return out


# speedup vs baseline: 1.1748x; 1.1748x over previous
import functools

import jax
import jax.numpy as jnp
from jax import lax
from jax.experimental import pallas as pl
from jax.experimental.pallas import tpu as pltpu
from jax.experimental.pallas import tpu_sc as plsc

N = 10000
E = 320000
C = 128
IC = 256
CH = C // 2

NC = 2
NS = 16
NW = NC * NS
WIN = 80

EPW_D = E // NW
NWIN_D = EPW_D // WIN

EPW_P = E // NS
WINP = 128
NFULL = EPW_P // WINP
TAIL = EPW_P - NFULL * WINP
NBUF = 5
NGRP = NFULL // NBUF

_MESH = plsc.VectorSubcoreMesh(
    core_axis_name="c", subcore_axis_name="s", num_cores=NC, num_subcores=NS)

_Z16 = functools.partial(jnp.zeros, (16,), jnp.float32)


def _deg_body(dst_hbm, out_hbm, idxb, ones_v, zb, deg_sh):
    cid = lax.axis_index("c")
    sid = lax.axis_index("s")
    wid = cid * NS + sid

    def fill(i, _):
        ones_v[pl.ds(i * 16, 16)] = _Z16() + 1.0
        return 0
    lax.fori_loop(0, WIN // 16, fill, 0)

    def fillz(i, _):
        zb[pl.ds(i * 16, 16)] = _Z16()
        return 0
    lax.fori_loop(0, 2000 // 16, fillz, 0)

    @pl.when(sid < 5)
    def _():
        pltpu.sync_copy(zb, deg_sh.at[pl.ds(sid * 2000, 2000)])

    pltpu.sync_copy(dst_hbm.at[wid], idxb)
    plsc.subcore_barrier()

    def body(w, _):
        pltpu.sync_copy(ones_v, deg_sh.at[idxb.at[w]], add=True)
        return 0
    lax.fori_loop(0, NWIN_D, body, 0)

    plsc.subcore_barrier()

    @pl.when(sid < 5)
    def _():
        pltpu.sync_copy(deg_sh.at[pl.ds(sid * 2000, 2000)], zb)
        pltpu.sync_copy(zb, out_hbm.at[pl.ds(cid * N + sid * 2000, 2000)])


_deg_kernel = pl.kernel(
    _deg_body,
    out_type=jax.ShapeDtypeStruct((NC * N,), jnp.float32),
    mesh=_MESH,
    scratch_types=[
        pltpu.VMEM((NWIN_D, WIN), jnp.int32),
        pltpu.VMEM((WIN,), jnp.float32),
        pltpu.VMEM((2000,), jnp.float32),
        pltpu.VMEM_SHARED((N,), jnp.float32),
    ],
)


def _prop_body(u_hbm, es_hbm, out_hbm, srcb, dstb, rows, acc_sh, *sems):
    gsem = sems[:NBUF]
    ssem = sems[NBUF:]
    cid = lax.axis_index("c")
    sid = lax.axis_index("s")

    def zfill(i, _):
        for cc in range(CH // 16):
            rows[0, i, pl.ds(cc * 16, 16)] = _Z16()
        return 0
    lax.fori_loop(0, WINP, zfill, 0)

    for k in range(5):
        c = sid + k * NS

        @pl.when(c < 78)
        def _():
            pltpu.sync_copy(rows.at[0], acc_sh.at[pl.ds(c * WINP, WINP)])

    @pl.when(sid == 0)
    def _():
        pltpu.sync_copy(rows.at[0, pl.ds(0, 16)], acc_sh.at[pl.ds(9984, 16)])

    e0 = sid * EPW_P
    pltpu.sync_copy(es_hbm.at[0, pl.ds(e0, EPW_P)], srcb)
    pltpu.sync_copy(es_hbm.at[1, pl.ds(e0, EPW_P)], dstb)
    plsc.subcore_barrier()

    uh = u_hbm.at[cid]

    def gather(w, b, n=WINP):
        pltpu.async_copy(uh.at[srcb.at[pl.ds(w * WINP, n)]],
                         rows.at[b, pl.ds(0, n)] if n != WINP else rows.at[b],
                         gsem[b])

    def gwait(w, b, n=WINP):
        pltpu.make_async_copy(uh.at[srcb.at[pl.ds(w * WINP, n)]],
                              rows.at[b, pl.ds(0, n)] if n != WINP
                              else rows.at[b], gsem[b]).wait()

    def scatter(w, b, n=WINP):
        pltpu.async_copy(rows.at[b, pl.ds(0, n)] if n != WINP
                         else rows.at[b],
                         acc_sh.at[dstb.at[pl.ds(w * WINP, n)]], ssem[b],
                         add=True)

    def swait(w, b, n=WINP):
        pltpu.make_async_copy(rows.at[b, pl.ds(0, n)] if n != WINP
                              else rows.at[b],
                              acc_sh.at[dstb.at[pl.ds(w * WINP, n)]],
                              ssem[b]).wait()

    for b in range(NBUF):
        gather(b, b)

    def group(g, _):
        for b in range(NBUF):
            w = g * NBUF + b
            gwait(w, b)
            scatter(w, b)

            @pl.when(w + NBUF < NFULL)
            def _():
                swait(w, b)
                gather(w + NBUF, b)
        return 0

    lax.fori_loop(0, NGRP, group, 0)

    w155 = NGRP * NBUF
    gwait(w155, 0)
    scatter(w155, 0)

    swait(w155 - NBUF + 1, 1)
    gather(NFULL, 1, TAIL)
    gwait(NFULL, 1, TAIL)
    scatter(NFULL, 1, TAIL)

    swait(w155, 0)
    swait(NFULL, 1, TAIL)
    for b in range(2, NBUF):
        swait((NGRP - 1) * NBUF + b, b)

    plsc.subcore_barrier()
    for k in range(5):
        c = sid + k * NS

        @pl.when(c < 78)
        def _():
            pltpu.sync_copy(acc_sh.at[pl.ds(c * WINP, WINP)], rows.at[k])
            pltpu.sync_copy(rows.at[k], out_hbm.at[cid, pl.ds(c * WINP, WINP)])

    @pl.when(sid == 0)
    def _():
        pltpu.sync_copy(acc_sh.at[pl.ds(9984, 16)], rows.at[0, pl.ds(0, 16)])
        pltpu.sync_copy(rows.at[0, pl.ds(0, 16)],
                        out_hbm.at[cid, pl.ds(9984, 16)])


_prop_kernel = pl.kernel(
    _prop_body,
    out_type=jax.ShapeDtypeStruct((NC, N, CH), jnp.float32),
    mesh=_MESH,
    scratch_types=[
        pltpu.VMEM((EPW_P,), jnp.int32),
        pltpu.VMEM((EPW_P,), jnp.int32),
        pltpu.VMEM((NBUF, WINP, CH), jnp.float32),
        pltpu.VMEM_SHARED((N, CH), jnp.float32),
    ] + [pltpu.SemaphoreType.DMA] * (2 * NBUF),
    compiler_params=pltpu.CompilerParams(use_tc_tiling_on_sc=False),
)


def _pre_body(degp_ref, x_ref, dinv_ref, invdeg_ref, u1_ref):
    deg = degp_ref[0] + degp_ref[1] + 1.0
    dinv = lax.rsqrt(deg)
    dinv_ref[...] = dinv
    invdeg_ref[...] = 1.0 / deg
    u1 = x_ref[...] * dinv
    u1_ref[0] = u1[:, :CH]
    u1_ref[1] = u1[:, CH:]


def _mid_body(agg_ref, x_ref, dinv_ref, invdeg_ref, w1_ref, g1_ref, b1_ref,
              w2_ref, t_ref, u2_ref):
    agg = jnp.concatenate([agg_ref[0], agg_ref[1]], axis=1)
    s1 = agg * dinv_ref[...] + x_ref[...] * invdeg_ref[...]
    h1 = jnp.dot(s1, w1_ref[...], preferred_element_type=jnp.float32)
    m = jnp.mean(h1, axis=0, keepdims=True)
    d = h1 - m
    v = jnp.mean(d * d, axis=0, keepdims=True)
    hn = jnp.maximum(g1_ref[...] * d / jnp.sqrt(v + 1e-5) + b1_ref[...], 0.0)
    t = jnp.dot(hn, w2_ref[...], preferred_element_type=jnp.float32)
    t_ref[...] = t
    u2 = t * dinv_ref[...]
    u2_ref[0] = u2[:, :CH]
    u2_ref[1] = u2[:, CH:]


def _post_body(agg_ref, t_ref, x_ref, dinv_ref, invdeg_ref, g2_ref, b2_ref,
               out_ref):
    agg = jnp.concatenate([agg_ref[0], agg_ref[1]], axis=1)
    s2 = agg * dinv_ref[...] + t_ref[...] * invdeg_ref[...]
    m = jnp.mean(s2, axis=0, keepdims=True)
    d = s2 - m
    v = jnp.mean(d * d, axis=0, keepdims=True)
    bn = g2_ref[...] * d / jnp.sqrt(v + 1e-5) + b2_ref[...]
    out_ref[...] = jnp.maximum(bn + x_ref[...], 0.0)


def _tc_call(body, out_shapes):
    return pl.pallas_call(body, out_shape=out_shapes)


def kernel(x, es, W1, W2, g1, b1, g2, b2):
    esd = es.reshape(2, NW, NWIN_D, WIN)

    degp = _deg_kernel(esd[1])
    degp = degp.reshape(NC, N, 1)

    f32 = jnp.float32
    dinv, invdeg, u1 = _tc_call(_pre_body, [
        jax.ShapeDtypeStruct((N, 1), f32),
        jax.ShapeDtypeStruct((N, 1), f32),
        jax.ShapeDtypeStruct((NC, N, CH), f32),
    ])(degp, x)

    agg1 = _prop_kernel(u1, es)

    t, u2 = _tc_call(_mid_body, [
        jax.ShapeDtypeStruct((N, C), f32),
        jax.ShapeDtypeStruct((NC, N, CH), f32),
    ])(agg1, x, dinv, invdeg, W1, g1.reshape(1, IC), b1.reshape(1, IC), W2)

    agg2 = _prop_kernel(u2, es)

    out = _tc_call(_post_body, jax.ShapeDtypeStruct((N, C), f32))(
        agg2, t, x, dinv, invdeg, g2.reshape(1, C), b2.reshape(1, C))
    return out
